# baseline (speedup 1.0000x reference)
import functools

import jax
import jax.numpy as jnp
from jax import lax
from jax.experimental import pallas as pl
from jax.experimental.pallas import tpu as pltpu
from jax.experimental.pallas import tpu_sc as plsc

NC = 2
NS = 16
NW = NC * NS
LANES = 16

DIM = 64
NEI = 9
NEI_PAD = 16
CR = 256


def _sc_gather_pool(ids, r_ids, neigh16, ent_emb, rel_emb):
    two_b = ids.shape[0]
    b = two_b // 2
    s_half = b // NW
    rel_per_w = b // NW
    max_rows = s_half * NEI
    cap = max_rows + CR
    max_ch = cap // CR

    mesh = plsc.VectorSubcoreMesh(core_axis_name="c", subcore_axis_name="s")

    @functools.partial(
        pl.kernel,
        out_type=(
            jax.ShapeDtypeStruct((b, DIM), jnp.float32),
            jax.ShapeDtypeStruct((b, DIM), jnp.float32),
            jax.ShapeDtypeStruct((b, DIM), jnp.float32),
        ),
        mesh=mesh,
        scratch_types=[
            pltpu.VMEM((s_half,), jnp.int32),
            pltpu.VMEM((s_half,), jnp.int32),
            pltpu.VMEM((rel_per_w // 2,), jnp.int32),
            pltpu.VMEM((rel_per_w // 2,), jnp.int32),
            pltpu.VMEM((s_half, NEI_PAD), jnp.int32),
            pltpu.VMEM((cap,), jnp.int32),
            pltpu.VMEM((cap,), jnp.int32),
            pltpu.VMEM((1,), jnp.int32),
            pltpu.VMEM((CR, DIM), jnp.float32),
            pltpu.VMEM((CR, DIM), jnp.float32),
            pltpu.VMEM((s_half, DIM), jnp.float32),
            pltpu.VMEM((rel_per_w // 2, DIM), jnp.float32),
            pltpu.SemaphoreType.DMA,
            pltpu.SemaphoreType.DMA,
            pltpu.SemaphoreType.DMA,
        ],
        compiler_params=pltpu.CompilerParams(use_tc_tiling_on_sc=False,
                                             needs_layout_passes=False,
                                             disable_bounds_checks=True),
    )
    def k(ids_hbm, rid_hbm, neigh_hbm, ent_hbm, rel_hbm,
          hsum_out, tsum_out, rrow_out,
          hid_v, tid_v, rid0_v, rid1_v, neigh_v, ci_v, st_v, cnt_v,
          e0_v, e1_v, acc_v, rrow_v, sem_n, sem_e0, sem_e1):
        wid = lax.axis_index("s") * NC + lax.axis_index("c")
        base = wid * s_half
        rel_half = rel_per_w // 2

        pltpu.sync_copy(ids_hbm.at[pl.ds(base, s_half)], hid_v)
        pltpu.sync_copy(ids_hbm.at[pl.ds(b + base, s_half)], tid_v)
        pltpu.sync_copy(rid_hbm.at[pl.ds(wid * rel_per_w, rel_half)], rid0_v)
        pltpu.sync_copy(
            rid_hbm.at[pl.ds(wid * rel_per_w + rel_half, rel_half)], rid1_v)

        lane = lax.iota(jnp.int32, LANES)
        zeros16 = jnp.zeros((LANES,), jnp.int32)
        e_v = (e0_v, e1_v)
        sem_e = (sem_e0, sem_e1)

        def do_side(id_v, out_hbm):
            pltpu.async_copy(neigh_hbm.at[id_v], neigh_v, sem_n).wait()

            def zslot(g, carry):
                for q in range(DIM // LANES):
                    acc_v[g, pl.ds(q * LANES, LANES)] = jnp.zeros(
                        (LANES,), jnp.float32)
                return carry

            lax.fori_loop(0, s_half, zslot, 0)

            def comp(i, off):
                rows = i * LANES + lane
                for j in range(NEI):
                    v = plsc.load_gather(
                        neigh_v, [rows, jnp.full((LANES,), j, jnp.int32)])
                    m = v > 0
                    pos = off + plsc.cumsum(jnp.where(m, 1, 0)) - 1
                    plsc.store_scatter(ci_v, [pos], v, mask=m)
                    plsc.store_scatter(st_v, [pos], rows, mask=m)
                    off = off + lax.reduce_sum(jnp.where(m, 1, 0), axes=(0,))
                return off

            nrows = lax.fori_loop(0, s_half // LANES, comp, jnp.int32(0))

            for t in range(CR // LANES):
                plsc.store_scatter(ci_v, [nrows + t * LANES + lane], zeros16)
                plsc.store_scatter(st_v, [nrows + t * LANES + lane], zeros16)
            nch = (nrows + CR - 1) // CR

            def issue(chunk, p):
                @pl.when(chunk < nch)
                def _():
                    pltpu.async_copy(
                        ent_hbm.at[ci_v.at[pl.ds(chunk * CR, CR)]],
                        e_v[p], sem_e[p])

            def process(chunk, p):
                @pl.when(chunk < nch)
                def _():
                    pltpu.make_async_copy(
                        ent_hbm.at[ci_v.at[pl.ds(chunk * CR, CR)]],
                        e_v[p], sem_e[p]).wait()

                    def row16(rr, carry):
                        tags = st_v[pl.ds(chunk * CR + rr * LANES, LANES)]
                        for l in range(LANES):
                            s = tags[l]
                            r = rr * LANES + l
                            for q in range(DIM // LANES):
                                sl = pl.ds(q * LANES, LANES)
                                acc_v[s, sl] = acc_v[s, sl] + e_v[p][r, sl]
                        return carry

                    lax.fori_loop(0, CR // LANES, row16, 0)

            issue(0, 0)
            issue(1, 1)

            def body(kk, carry):
                chunk = 2 * kk
                process(chunk, 0)
                issue(chunk + 2, 0)
                process(chunk + 1, 1)
                issue(chunk + 3, 1)
                return carry

            lax.fori_loop(0, max_ch // 2, body, 0)

            pltpu.sync_copy(acc_v, out_hbm.at[pl.ds(base, s_half)])

        do_side(hid_v, hsum_out)
        do_side(tid_v, tsum_out)

        pltpu.async_copy(rel_hbm.at[rid0_v], rrow_v, sem_n).wait()
        pltpu.sync_copy(rrow_v, rrow_out.at[pl.ds(wid * rel_per_w, rel_half)])
        pltpu.async_copy(rel_hbm.at[rid1_v], rrow_v, sem_n).wait()
        pltpu.sync_copy(
            rrow_v, rrow_out.at[pl.ds(wid * rel_per_w + rel_half, rel_half)])

    return k(ids, r_ids, neigh16, ent_emb, rel_emb)


def _tc_score(hsum, rrow, tsum):
    b = hsum.shape[0]
    blk = 2048

    def body(h_ref, r_ref, t_ref, o_ref):
        def nrm(x):
            n2 = jnp.sum(x * x, axis=1, keepdims=True)
            return x / jnp.maximum(jnp.sqrt(n2), 1e-12)

        v = nrm(h_ref[...]) + nrm(r_ref[...]) - nrm(t_ref[...])
        o_ref[...] = jnp.sum(jnp.abs(v), axis=1)

    return pl.pallas_call(
        body,
        grid=(b // blk,),
        in_specs=[
            pl.BlockSpec((blk, DIM), lambda i: (i, 0)),
            pl.BlockSpec((blk, DIM), lambda i: (i, 0)),
            pl.BlockSpec((blk, DIM), lambda i: (i, 0)),
        ],
        out_specs=pl.BlockSpec((blk,), lambda i: (i,)),
        out_shape=jax.ShapeDtypeStruct((b,), jnp.float32),
    )(hsum, rrow, tsum)


def _tc_pad16(neigh_table):
    n = neigh_table.shape[0]
    br = 16384

    def body(x_ref, o_ref):
        o_ref[...] = jnp.concatenate(
            [x_ref[...], jnp.zeros((br, NEI_PAD - NEI), jnp.int32)], axis=1)

    return pl.pallas_call(
        body,
        grid=(n // br,),
        in_specs=[pl.BlockSpec((br, NEI), lambda i: (i, 0))],
        out_specs=pl.BlockSpec((br, NEI_PAD), lambda i: (i, 0)),
        out_shape=jax.ShapeDtypeStruct((n, NEI_PAD), jnp.int32),
    )(neigh_table)


def kernel(triples, ent_emb, rel_emb, neigh_table, neigh_lens):
    del neigh_lens
    h_ids = triples[:, 0]
    r_ids = triples[:, 1]
    t_ids = triples[:, 2]
    ids = jnp.concatenate([h_ids, t_ids], axis=0)
    neigh16 = _tc_pad16(neigh_table)
    hsum, tsum, rrow = _sc_gather_pool(ids, r_ids, neigh16, ent_emb, rel_emb)
    return _tc_score(hsum, rrow, tsum)


# speedup vs baseline: 1.0615x; 1.0615x over previous
import functools

import jax
import jax.numpy as jnp
from jax import lax
from jax.experimental import pallas as pl
from jax.experimental.pallas import tpu as pltpu
from jax.experimental.pallas import tpu_sc as plsc

NC = 2
NS = 16
NW = NC * NS
LANES = 16

DIM = 64
NEI = 9
NEI_PAD = 16
CR = 256


def _sc_gather_pool(ids, r_ids, neigh16, ent_emb, rel_emb):
    two_b = ids.shape[0]
    b = two_b // 2
    s_half = b // NW
    rel_per_w = b // NW
    max_rows = s_half * NEI
    cap = max_rows + CR
    max_ch = cap // CR

    mesh = plsc.VectorSubcoreMesh(core_axis_name="c", subcore_axis_name="s")

    @functools.partial(
        pl.kernel,
        out_type=(
            jax.ShapeDtypeStruct((b, DIM), jnp.float32),
            jax.ShapeDtypeStruct((b, DIM), jnp.float32),
            jax.ShapeDtypeStruct((b, DIM), jnp.float32),
        ),
        mesh=mesh,
        scratch_types=[
            pltpu.VMEM((s_half,), jnp.int32),
            pltpu.VMEM((s_half,), jnp.int32),
            pltpu.VMEM((rel_per_w // 2,), jnp.int32),
            pltpu.VMEM((rel_per_w // 2,), jnp.int32),
            pltpu.VMEM((s_half, NEI_PAD), jnp.int32),
            pltpu.VMEM((cap,), jnp.int32),
            pltpu.VMEM((cap,), jnp.int32),
            pltpu.VMEM((1,), jnp.int32),
            pltpu.VMEM((CR, DIM), jnp.float32),
            pltpu.VMEM((CR, DIM), jnp.float32),
            pltpu.VMEM((s_half, DIM), jnp.float32),
            pltpu.VMEM((rel_per_w // 2, DIM), jnp.float32),
            pltpu.SemaphoreType.DMA,
            pltpu.SemaphoreType.DMA,
            pltpu.SemaphoreType.DMA,
        ],
        compiler_params=pltpu.CompilerParams(use_tc_tiling_on_sc=False,
                                             needs_layout_passes=False,
                                             disable_bounds_checks=True),
    )
    def k(ids_hbm, rid_hbm, neigh_hbm, ent_hbm, rel_hbm,
          hsum_out, tsum_out, rrow_out,
          hid_v, tid_v, rid0_v, rid1_v, neigh_v, ci_v, st_v, cnt_v,
          e0_v, e1_v, acc_v, rrow_v, sem_n, sem_e0, sem_e1):
        wid = lax.axis_index("s") * NC + lax.axis_index("c")
        base = wid * s_half
        rel_half = rel_per_w // 2

        pltpu.sync_copy(ids_hbm.at[pl.ds(base, s_half)], hid_v)
        pltpu.sync_copy(ids_hbm.at[pl.ds(b + base, s_half)], tid_v)
        pltpu.sync_copy(rid_hbm.at[pl.ds(wid * rel_per_w, rel_half)], rid0_v)
        pltpu.sync_copy(
            rid_hbm.at[pl.ds(wid * rel_per_w + rel_half, rel_half)], rid1_v)

        lane = lax.iota(jnp.int32, LANES)
        zeros16 = jnp.zeros((LANES,), jnp.int32)
        e_v = (e0_v, e1_v)
        sem_e = (sem_e0, sem_e1)

        def do_side(id_v, out_hbm):
            pltpu.async_copy(neigh_hbm.at[id_v], neigh_v, sem_n).wait()

            def zslot(g, carry):
                for q in range(DIM // LANES):
                    acc_v[g, pl.ds(q * LANES, LANES)] = jnp.zeros(
                        (LANES,), jnp.float32)
                return carry

            lax.fori_loop(0, s_half, zslot, 0)

            def comp(i, off):
                rows = i * LANES + lane
                for j in range(NEI):
                    v = plsc.load_gather(
                        neigh_v, [rows, jnp.full((LANES,), j, jnp.int32)])
                    m = v > 0
                    pos = off + plsc.cumsum(jnp.where(m, 1, 0)) - 1
                    plsc.store_scatter(ci_v, [pos], v, mask=m)
                    plsc.store_scatter(st_v, [pos], rows, mask=m)
                    off = off + lax.reduce_sum(jnp.where(m, 1, 0), axes=(0,))
                return off

            nrows = lax.fori_loop(0, s_half // LANES, comp, jnp.int32(0))

            for t in range(CR // LANES):
                plsc.store_scatter(ci_v, [nrows + t * LANES + lane], zeros16)
                plsc.store_scatter(st_v, [nrows + t * LANES + lane], zeros16)
            nch = (nrows + CR - 1) // CR

            def issue(chunk, p):
                @pl.when(chunk < nch)
                def _():
                    pltpu.async_copy(
                        ent_hbm.at[ci_v.at[pl.ds(chunk * CR, CR)]],
                        e_v[p], sem_e[p])

            def process(chunk, p):
                @pl.when(chunk < nch)
                def _():
                    pltpu.make_async_copy(
                        ent_hbm.at[ci_v.at[pl.ds(chunk * CR, CR)]],
                        e_v[p], sem_e[p]).wait()

                    def row16(rr, carry):
                        tags = st_v[pl.ds(chunk * CR + rr * LANES, LANES)]
                        for l in range(LANES):
                            s = tags[l]
                            r = rr * LANES + l
                            for q in range(DIM // LANES):
                                sl = pl.ds(q * LANES, LANES)
                                acc_v[s, sl] = acc_v[s, sl] + e_v[p][r, sl]
                        return carry

                    lax.fori_loop(0, CR // LANES, row16, 0)

            issue(0, 0)
            issue(1, 1)

            def body(kk, carry):
                chunk = 2 * kk
                process(chunk, 0)
                issue(chunk + 2, 0)
                process(chunk + 1, 1)
                issue(chunk + 3, 1)
                return carry

            lax.fori_loop(0, max_ch // 2, body, 0)

            pltpu.sync_copy(acc_v, out_hbm.at[pl.ds(base, s_half)])

        do_side(hid_v, hsum_out)
        do_side(tid_v, tsum_out)

        pltpu.async_copy(rel_hbm.at[rid0_v], rrow_v, sem_n).wait()
        pltpu.sync_copy(rrow_v, rrow_out.at[pl.ds(wid * rel_per_w, rel_half)])
        pltpu.async_copy(rel_hbm.at[rid1_v], rrow_v, sem_n).wait()
        pltpu.sync_copy(
            rrow_v, rrow_out.at[pl.ds(wid * rel_per_w + rel_half, rel_half)])

    return k(ids, r_ids, neigh16, ent_emb, rel_emb)


def _tc_score(hsum, rrow, tsum):
    b = hsum.shape[0]
    blk = 2048

    def body(h_ref, r_ref, t_ref, o_ref):
        def nrm(x):
            n2 = jnp.sum(x * x, axis=1, keepdims=True)
            return x / jnp.maximum(jnp.sqrt(n2), 1e-12)

        v = nrm(h_ref[...]) + nrm(r_ref[...]) - nrm(t_ref[...])
        o_ref[...] = jnp.sum(jnp.abs(v), axis=1)

    return pl.pallas_call(
        body,
        grid=(b // blk,),
        in_specs=[
            pl.BlockSpec((blk, DIM), lambda i: (i, 0)),
            pl.BlockSpec((blk, DIM), lambda i: (i, 0)),
            pl.BlockSpec((blk, DIM), lambda i: (i, 0)),
        ],
        out_specs=pl.BlockSpec((blk,), lambda i: (i,)),
        out_shape=jax.ShapeDtypeStruct((b,), jnp.float32),
    )(hsum, rrow, tsum)


def kernel(triples, ent_emb, rel_emb, neigh_table, neigh_lens):
    del neigh_lens
    h_ids = triples[:, 0]
    r_ids = triples[:, 1]
    t_ids = triples[:, 2]
    ids = jnp.concatenate([h_ids, t_ids], axis=0)
    neigh16 = jnp.pad(neigh_table, ((0, 0), (0, NEI_PAD - NEI)))
    hsum, tsum, rrow = _sc_gather_pool(ids, r_ids, neigh16, ent_emb, rel_emb)
    return _tc_score(hsum, rrow, tsum)


# speedup vs baseline: 1.0989x; 1.0353x over previous
import functools

import jax
import jax.numpy as jnp
from jax import lax
from jax.experimental import pallas as pl
from jax.experimental.pallas import tpu as pltpu
from jax.experimental.pallas import tpu_sc as plsc

NC = 2
NS = 16
NW = NC * NS
LANES = 16

DIM = 64
NEI = 9
NEI_PAD = 16
CR = 384


def _sc_gather_pool(ids, r_ids, neigh16, ent_emb, rel_emb):
    two_b = ids.shape[0]
    b = two_b // 2
    s_half = b // NW
    rel_per_w = b // NW
    max_rows = s_half * NEI
    cap = max_rows + CR
    max_ch = cap // CR

    mesh = plsc.VectorSubcoreMesh(core_axis_name="c", subcore_axis_name="s")

    @functools.partial(
        pl.kernel,
        out_type=(
            jax.ShapeDtypeStruct((b, DIM), jnp.float32),
            jax.ShapeDtypeStruct((b, DIM), jnp.float32),
            jax.ShapeDtypeStruct((b, DIM), jnp.float32),
        ),
        mesh=mesh,
        scratch_types=[
            pltpu.VMEM((s_half,), jnp.int32),
            pltpu.VMEM((s_half,), jnp.int32),
            pltpu.VMEM((rel_per_w // 2,), jnp.int32),
            pltpu.VMEM((rel_per_w // 2,), jnp.int32),
            pltpu.VMEM((s_half, NEI_PAD), jnp.int32),
            pltpu.VMEM((cap,), jnp.int32),
            pltpu.VMEM((cap,), jnp.int32),
            pltpu.VMEM((CR, DIM), jnp.float32),
            pltpu.VMEM((CR, DIM), jnp.float32),
            pltpu.VMEM((s_half, DIM), jnp.float32),
            pltpu.VMEM((rel_per_w // 2, DIM), jnp.float32),
            pltpu.SemaphoreType.DMA,
            pltpu.SemaphoreType.DMA,
            pltpu.SemaphoreType.DMA,
        ],
        compiler_params=pltpu.CompilerParams(use_tc_tiling_on_sc=False,
                                             needs_layout_passes=False,
                                             disable_bounds_checks=True),
    )
    def k(ids_hbm, rid_hbm, neigh_hbm, ent_hbm, rel_hbm,
          hsum_out, tsum_out, rrow_out,
          hid_v, tid_v, rid0_v, rid1_v, neigh_v, ci_v, st_v,
          e0_v, e1_v, acc_v, rrow_v, sem_n, sem_e0, sem_e1):
        wid = lax.axis_index("s") * NC + lax.axis_index("c")
        base = wid * s_half
        rel_half = rel_per_w // 2

        pltpu.sync_copy(ids_hbm.at[pl.ds(base, s_half)], hid_v)
        pltpu.sync_copy(ids_hbm.at[pl.ds(b + base, s_half)], tid_v)
        pltpu.sync_copy(rid_hbm.at[pl.ds(wid * rel_per_w, rel_half)], rid0_v)
        pltpu.sync_copy(
            rid_hbm.at[pl.ds(wid * rel_per_w + rel_half, rel_half)], rid1_v)

        lane = lax.iota(jnp.int32, LANES)
        zeros16 = jnp.zeros((LANES,), jnp.int32)
        e_v = (e0_v, e1_v)
        sem_e = (sem_e0, sem_e1)

        def do_side(id_v, out_hbm):
            pltpu.async_copy(neigh_hbm.at[id_v], neigh_v, sem_n).wait()

            def zslot(g, carry):
                for q in range(DIM // LANES):
                    acc_v[g, pl.ds(q * LANES, LANES)] = jnp.zeros(
                        (LANES,), jnp.float32)
                return carry

            lax.fori_loop(0, s_half, zslot, 0)

            def comp(i, off):
                rows = i * LANES + lane
                for j in range(NEI):
                    v = plsc.load_gather(
                        neigh_v, [rows, jnp.full((LANES,), j, jnp.int32)])
                    m = v > 0
                    pos = off + plsc.cumsum(jnp.where(m, 1, 0)) - 1
                    plsc.store_scatter(ci_v, [pos], v, mask=m)
                    plsc.store_scatter(st_v, [pos], rows, mask=m)
                    off = off + lax.reduce_sum(jnp.where(m, 1, 0), axes=(0,))
                return off

            nrows = lax.fori_loop(0, s_half // LANES, comp, jnp.int32(0))

            for t in range(CR // LANES):
                plsc.store_scatter(ci_v, [nrows + t * LANES + lane], zeros16)
                plsc.store_scatter(st_v, [nrows + t * LANES + lane], zeros16)
            nch = (nrows + CR - 1) // CR

            def issue(chunk, p):
                @pl.when(chunk < nch)
                def _():
                    pltpu.async_copy(
                        ent_hbm.at[ci_v.at[pl.ds(chunk * CR, CR)]],
                        e_v[p], sem_e[p])

            def process(chunk, p):
                @pl.when(chunk < nch)
                def _():
                    pltpu.make_async_copy(
                        ent_hbm.at[ci_v.at[pl.ds(chunk * CR, CR)]],
                        e_v[p], sem_e[p]).wait()

                    def row16(rr, carry):
                        tags = st_v[pl.ds(chunk * CR + rr * LANES, LANES)]
                        for l in range(LANES):
                            s = tags[l]
                            r = rr * LANES + l
                            for q in range(DIM // LANES):
                                sl = pl.ds(q * LANES, LANES)
                                acc_v[s, sl] = acc_v[s, sl] + e_v[p][r, sl]
                        return carry

                    lax.fori_loop(0, CR // LANES, row16, 0)

            issue(0, 0)
            issue(1, 1)

            def body(kk, carry):
                chunk = 2 * kk
                process(chunk, 0)
                issue(chunk + 2, 0)
                process(chunk + 1, 1)
                issue(chunk + 3, 1)
                return carry

            lax.fori_loop(0, max_ch // 2, body, 0)

            pltpu.sync_copy(acc_v, out_hbm.at[pl.ds(base, s_half)])

        do_side(hid_v, hsum_out)
        do_side(tid_v, tsum_out)

        pltpu.async_copy(rel_hbm.at[rid0_v], rrow_v, sem_n).wait()
        pltpu.sync_copy(rrow_v, rrow_out.at[pl.ds(wid * rel_per_w, rel_half)])
        pltpu.async_copy(rel_hbm.at[rid1_v], rrow_v, sem_n).wait()
        pltpu.sync_copy(
            rrow_v, rrow_out.at[pl.ds(wid * rel_per_w + rel_half, rel_half)])

    return k(ids, r_ids, neigh16, ent_emb, rel_emb)


def _tc_score(hsum, rrow, tsum):
    b = hsum.shape[0]
    blk = 2048

    def body(h_ref, r_ref, t_ref, o_ref):
        def nrm(x):
            n2 = jnp.sum(x * x, axis=1, keepdims=True)
            return x / jnp.maximum(jnp.sqrt(n2), 1e-12)

        v = nrm(h_ref[...]) + nrm(r_ref[...]) - nrm(t_ref[...])
        o_ref[...] = jnp.sum(jnp.abs(v), axis=1)

    return pl.pallas_call(
        body,
        grid=(b // blk,),
        in_specs=[
            pl.BlockSpec((blk, DIM), lambda i: (i, 0)),
            pl.BlockSpec((blk, DIM), lambda i: (i, 0)),
            pl.BlockSpec((blk, DIM), lambda i: (i, 0)),
        ],
        out_specs=pl.BlockSpec((blk,), lambda i: (i,)),
        out_shape=jax.ShapeDtypeStruct((b,), jnp.float32),
    )(hsum, rrow, tsum)


def kernel(triples, ent_emb, rel_emb, neigh_table, neigh_lens):
    del neigh_lens
    h_ids = triples[:, 0]
    r_ids = triples[:, 1]
    t_ids = triples[:, 2]
    ids = jnp.concatenate([h_ids, t_ids], axis=0)
    neigh16 = jnp.pad(neigh_table, ((0, 0), (0, NEI_PAD - NEI)))
    hsum, tsum, rrow = _sc_gather_pool(ids, r_ids, neigh16, ent_emb, rel_emb)
    return _tc_score(hsum, rrow, tsum)
